# Initial kernel scaffold; baseline (speedup 1.0000x reference)
import jax
import jax.numpy as jnp
from jax.experimental import pallas as pl


def kernel(x, W, b):
    raise NotImplementedError("write your pallas kernel here")


# speedup vs baseline: 4.4730x; 4.4730x over previous
import jax
import jax.numpy as jnp
from jax import lax
from jax.experimental import pallas as pl
from jax.experimental.pallas import tpu as pltpu
from jax.experimental.pallas import tpu_sc as plsc

BATCH = 4096
NDIMS = 4
DIM = 1000
NW = 32
RPW = BATCH // NW
RW = 8
NWAVE = RPW // RW
L = 16
NFULL = DIM // L
TAIL_OFF = DIM - L

DMAP = (0, 0, 0, 1, 1, 2)
KMAP = (1, 2, 3, 2, 3, 3)
PICKS = ((0, 1), (1, 2), (3, 2), (2, 3), (4, 3), (5, 3))


def _sc_kernel(xT_hbm, wr_hbm, b_hbm, out_hbm,
               xall_v, idx_v, b_v, buf_v, svec_v, sem):
    wid = lax.axis_index("s") * 2 + lax.axis_index("c")
    base = wid * RPW

    for d in range(NDIMS):
        pltpu.sync_copy(xT_hbm.at[d, pl.ds(base, RPW)], xall_v.at[d, pl.ds(0, RPW)])
    pltpu.sync_copy(b_hbm.at[pl.ds(DIM, 3 * DIM)], b_v)

    iota = lax.iota(jnp.int32, L)

    for t in range(6):
        for kc in range(RPW // L):
            xc = xall_v[DMAP[t], pl.ds(kc * L, L)]
            idx_v[pl.ds(t * RPW + kc * L, L)] = \
                (xc + DIM * DMAP[t]) * 4 + KMAP[t]

    tailmask = iota >= (L - (DIM - NFULL * L))
    zero16 = jnp.zeros((L,), jnp.float32)

    def wave_body(w, carry):
        handles = [
            pltpu.async_copy(
                wr_hbm.at[idx_v.at[pl.ds(t * RPW + w * RW, RW)]],
                buf_v.at[t], sem)
            for t in range(6)
        ]
        for h in handles:
            h.wait()

        def chunk(off, accs, masked):
            b1 = b_v[pl.ds(off, L)]
            b2 = b_v[pl.ds(DIM + off, L)]
            b3 = b_v[pl.ds(2 * DIM + off, L)]
            new = []
            for r in range(RW):
                a1 = buf_v[0, r, pl.ds(off, L)]
                a2 = buf_v[1, r, pl.ds(off, L)]
                a3 = buf_v[2, r, pl.ds(off, L)]
                f2 = buf_v[3, r, pl.ds(off, L)]
                f3 = buf_v[4, r, pl.ds(off, L)]
                g3 = buf_v[5, r, pl.ds(off, L)]
                e1 = jnp.exp(a1 + b1)
                e2 = jnp.exp(a2 + f2 + b2)
                e3 = jnp.exp(a3 + f3 + g3 + b3)
                if masked:
                    e1 = jnp.where(tailmask, e1, 0.0)
                    e2 = jnp.where(tailmask, e2, 0.0)
                    e3 = jnp.where(tailmask, e3, 0.0)
                s1, s2, s3 = accs[3 * r:3 * r + 3]
                new += [s1 + e1, s2 + e2, s3 + e3]
            return tuple(new)

        accs = lax.fori_loop(
            0, NFULL,
            lambda c, a: chunk(c * L, a, False),
            tuple([zero16] * (3 * RW)),
        )
        accs = chunk(TAIL_OFF, accs, True)

        for r in range(RW):
            row = w * RW + r
            xsc = [None] * NDIMS
            for j in (1, 2, 3):
                xv = xall_v[j, pl.ds(row, L)]
                xsc[j] = xv[0]
            selsum = 0.0
            for t, j in PICKS:
                col = xsc[j]
                rem = lax.rem(col, L)
                v = buf_v[t, r, pl.ds(col - rem, L)]
                selsum += jnp.sum(jnp.where(iota == rem, v, 0.0))
            svec_v[pl.ds(row * 64, L)] = jnp.full((L,), selsum, jnp.float32)
            for j in range(3):
                svec_v[pl.ds(row * 64 + (j + 1) * L, L)] = accs[3 * r + j]
        return carry

    lax.fori_loop(0, NWAVE, wave_body, 0)

    pltpu.sync_copy(svec_v, out_hbm.at[pl.ds(wid * (RPW * 64), RPW * 64)])


BT = 256


def _tc_kernel(svecs_ref, x_ref, b_ref, out_ref):
    sv = svecs_ref[...]
    selw = sv[:, 0:1]
    s1 = jnp.sum(sv[:, 16:32], axis=1, keepdims=True)
    s2 = jnp.sum(sv[:, 32:48], axis=1, keepdims=True)
    s3 = jnp.sum(sv[:, 48:64], axis=1, keepdims=True)
    b_row = b_ref[...]
    b0 = b_row[:, 0:DIM]
    s0 = jnp.sum(jnp.exp(b0))
    cid = lax.broadcasted_iota(jnp.int32, (BT, DIM), 1)
    x = x_ref[...]
    bsel = jnp.zeros((BT, 1), jnp.float32)
    for j in range(NDIMS):
        bj = b_row[:, j * DIM:(j + 1) * DIM]
        xj = x[:, j:j + 1]
        bsel = bsel + jnp.sum(jnp.where(cid == xj, bj, 0.0),
                              axis=1, keepdims=True)
    out_ref[...] = (selw + bsel - jnp.log(s0) - jnp.log(s1)
                    - jnp.log(s2) - jnp.log(s3))


@jax.jit
def kernel(x, W, b):
    x32 = x.astype(jnp.int32)
    xT = x32.T
    Wr = W.reshape(3 * NDIMS * DIM, DIM)

    mesh = plsc.VectorSubcoreMesh(core_axis_name="c", subcore_axis_name="s")
    svecs = pl.kernel(
        _sc_kernel,
        mesh=mesh,
        out_type=[jax.ShapeDtypeStruct((BATCH * 64,), jnp.float32)],
        scratch_types=[
            pltpu.VMEM((NDIMS, RPW + L), jnp.int32),
            pltpu.VMEM((6 * RPW,), jnp.int32),
            pltpu.VMEM((3 * DIM,), jnp.float32),
            pltpu.VMEM((6, RW, DIM), jnp.float32),
            pltpu.VMEM((RPW * 64,), jnp.float32),
            pltpu.SemaphoreType.DMA,
        ],
        compiler_params=pltpu.CompilerParams(use_tc_tiling_on_sc=False,
                                             needs_layout_passes=False),
    )(xT, Wr, b)[0]

    out = pl.pallas_call(
        _tc_kernel,
        grid=(BATCH // BT,),
        in_specs=[
            pl.BlockSpec((BT, 64), lambda i: (i, 0)),
            pl.BlockSpec((BT, NDIMS), lambda i: (i, 0)),
            pl.BlockSpec((1, NDIMS * DIM), lambda i: (0, 0)),
        ],
        out_specs=pl.BlockSpec((BT, 1), lambda i: (i, 0)),
        out_shape=jax.ShapeDtypeStruct((BATCH, 1), jnp.float32),
    )(svecs.reshape(BATCH, 64), x32, b.reshape(1, NDIMS * DIM))

    return out.reshape(BATCH)


# speedup vs baseline: 5.4024x; 1.2078x over previous
import jax
import jax.numpy as jnp
from jax import lax
from jax.experimental import pallas as pl
from jax.experimental.pallas import tpu as pltpu
from jax.experimental.pallas import tpu_sc as plsc

BATCH = 4096
NDIMS = 4
DIM = 1000
NW = 32
RPW = BATCH // NW
RW = 8
NWAVE = RPW // RW
L = 16
NFULL = DIM // L
TAIL_OFF = DIM - L

DMAP = (0, 0, 0, 1, 1, 2)
KMAP = (1, 2, 3, 2, 3, 3)
PICKS = ((0, 1), (1, 2), (3, 2), (2, 3), (4, 3), (5, 3))


def _sc_kernel(xT_hbm, wr_hbm, b_hbm, out_hbm,
               xall_v, idx_v, b_v, buf_v, svec_v, sem0, sem1):
    wid = lax.axis_index("s") * 2 + lax.axis_index("c")
    base = wid * RPW

    for d in range(NDIMS):
        pltpu.sync_copy(xT_hbm.at[d, pl.ds(base, RPW)], xall_v.at[d, pl.ds(0, RPW)])
    pltpu.sync_copy(b_hbm.at[pl.ds(DIM, 3 * DIM)], b_v)

    iota = lax.iota(jnp.int32, L)

    for t in range(6):
        for kc in range(RPW // L):
            xc = xall_v[DMAP[t], pl.ds(kc * L, L)]
            idx_v[pl.ds(t * RPW + kc * L, L)] = \
                (xc + DIM * DMAP[t]) * 4 + KMAP[t]

    tailmask = iota >= (L - (DIM - NFULL * L))
    zero16 = jnp.zeros((L,), jnp.float32)

    def issue(w, pb, sem):
        for t in range(6):
            pltpu.async_copy(
                wr_hbm.at[idx_v.at[pl.ds(t * RPW + w * RW, RW)]],
                buf_v.at[pb, t], sem)

    def drain(w, pb, sem):
        for t in range(6):
            pltpu.make_async_copy(
                wr_hbm.at[idx_v.at[pl.ds(t * RPW + w * RW, RW)]],
                buf_v.at[pb, t], sem).wait()

    def compute(w, pb):
        def chunk(off, accs, masked):
            b1 = b_v[pl.ds(off, L)]
            b2 = b_v[pl.ds(DIM + off, L)]
            b3 = b_v[pl.ds(2 * DIM + off, L)]
            new = []
            for r in range(RW):
                a1 = buf_v[pb, 0, r, pl.ds(off, L)]
                a2 = buf_v[pb, 1, r, pl.ds(off, L)]
                a3 = buf_v[pb, 2, r, pl.ds(off, L)]
                f2 = buf_v[pb, 3, r, pl.ds(off, L)]
                f3 = buf_v[pb, 4, r, pl.ds(off, L)]
                g3 = buf_v[pb, 5, r, pl.ds(off, L)]
                e1 = jnp.exp(a1 + b1)
                e2 = jnp.exp(a2 + f2 + b2)
                e3 = jnp.exp(a3 + f3 + g3 + b3)
                if masked:
                    e1 = jnp.where(tailmask, e1, 0.0)
                    e2 = jnp.where(tailmask, e2, 0.0)
                    e3 = jnp.where(tailmask, e3, 0.0)
                s1, s2, s3 = accs[3 * r:3 * r + 3]
                new += [s1 + e1, s2 + e2, s3 + e3]
            return tuple(new)

        accs = lax.fori_loop(
            0, NFULL,
            lambda c, a: chunk(c * L, a, False),
            tuple([zero16] * (3 * RW)),
        )
        accs = chunk(TAIL_OFF, accs, True)

        for r in range(RW):
            row = w * RW + r
            xsc = [None] * NDIMS
            for j in (1, 2, 3):
                xv = xall_v[j, pl.ds(row, L)]
                xsc[j] = xv[0]
            sel = zero16
            for t, j in PICKS:
                col = xsc[j]
                rem = lax.rem(col, L)
                v = buf_v[pb, t, r, pl.ds(col - rem, L)]
                sel = sel + jnp.where(iota == rem, v, 0.0)
            svec_v[pl.ds(row * 64, L)] = sel
            for j in range(3):
                svec_v[pl.ds(row * 64 + (j + 1) * L, L)] = accs[3 * r + j]

    issue(0, 0, sem0)

    def body2(i, carry):
        w = 2 * i
        issue(w + 1, 1, sem1)
        drain(w, 0, sem0)
        compute(w, 0)
        issue(w + 2, 0, sem0)
        drain(w + 1, 1, sem1)
        compute(w + 1, 1)
        return carry

    lax.fori_loop(0, NWAVE // 2 - 1, body2, 0)
    issue(NWAVE - 1, 1, sem1)
    drain(NWAVE - 2, 0, sem0)
    compute(NWAVE - 2, 0)
    drain(NWAVE - 1, 1, sem1)
    compute(NWAVE - 1, 1)

    pltpu.sync_copy(svec_v, out_hbm.at[pl.ds(wid * (RPW * 64), RPW * 64)])


BT = 256


def _tc_kernel(svecs_ref, x_ref, b_ref, out_ref):
    sv = svecs_ref[...]
    selw = jnp.sum(sv[:, 0:16], axis=1, keepdims=True)
    s1 = jnp.sum(sv[:, 16:32], axis=1, keepdims=True)
    s2 = jnp.sum(sv[:, 32:48], axis=1, keepdims=True)
    s3 = jnp.sum(sv[:, 48:64], axis=1, keepdims=True)
    b_row = b_ref[...]
    b0 = b_row[:, 0:DIM]
    s0 = jnp.sum(jnp.exp(b0))
    cid = lax.broadcasted_iota(jnp.int32, (BT, DIM), 1)
    x = x_ref[...]
    bsel = jnp.zeros((BT, 1), jnp.float32)
    for j in range(NDIMS):
        bj = b_row[:, j * DIM:(j + 1) * DIM]
        xj = x[:, j:j + 1]
        bsel = bsel + jnp.sum(jnp.where(cid == xj, bj, 0.0),
                              axis=1, keepdims=True)
    out_ref[...] = (selw + bsel - jnp.log(s0) - jnp.log(s1)
                    - jnp.log(s2) - jnp.log(s3))


@jax.jit
def kernel(x, W, b):
    x32 = x.astype(jnp.int32)
    xT = x32.T
    Wr = W.reshape(3 * NDIMS * DIM, DIM)

    mesh = plsc.VectorSubcoreMesh(core_axis_name="c", subcore_axis_name="s")
    svecs = pl.kernel(
        _sc_kernel,
        mesh=mesh,
        out_type=[jax.ShapeDtypeStruct((BATCH * 64,), jnp.float32)],
        scratch_types=[
            pltpu.VMEM((NDIMS, RPW + L), jnp.int32),
            pltpu.VMEM((6 * RPW,), jnp.int32),
            pltpu.VMEM((3 * DIM,), jnp.float32),
            pltpu.VMEM((2, 6, RW, DIM), jnp.float32),
            pltpu.VMEM((RPW * 64,), jnp.float32),
            pltpu.SemaphoreType.DMA,
            pltpu.SemaphoreType.DMA,
        ],
        compiler_params=pltpu.CompilerParams(use_tc_tiling_on_sc=False,
                                             needs_layout_passes=False),
    )(xT, Wr, b)[0]

    out = pl.pallas_call(
        _tc_kernel,
        grid=(BATCH // BT,),
        in_specs=[
            pl.BlockSpec((BT, 64), lambda i: (i, 0)),
            pl.BlockSpec((BT, NDIMS), lambda i: (i, 0)),
            pl.BlockSpec((1, NDIMS * DIM), lambda i: (0, 0)),
        ],
        out_specs=pl.BlockSpec((BT, 1), lambda i: (i, 0)),
        out_shape=jax.ShapeDtypeStruct((BATCH, 1), jnp.float32),
    )(svecs.reshape(BATCH, 64), x32, b.reshape(1, NDIMS * DIM))

    return out.reshape(BATCH)


# speedup vs baseline: 5.6909x; 1.0534x over previous
import jax
import jax.numpy as jnp
from jax import lax
from jax.experimental import pallas as pl
from jax.experimental.pallas import tpu as pltpu
from jax.experimental.pallas import tpu_sc as plsc

BATCH = 4096
NDIMS = 4
DIM = 1000
NW = 32
RPW = BATCH // NW
RW = 8
NWAVE = RPW // RW
L = 16
NFULL = DIM // L
TAIL_OFF = DIM - L

DMAP = (0, 0, 0, 1, 1, 2)
AMAP = (0, 0, 0, 1, 1, 2)
MUL = (3, 3, 3, 2, 2, 1)
ADD = (0, 1, 2, 0, 1, 0)
PICKS = ((0, 1), (1, 2), (3, 2), (2, 3), (4, 3), (5, 3))


def _sc_kernel(xT_hbm, a_hbm, bt_hbm, c_hbm, bias_hbm, out_hbm,
               xall_v, idx_v, b_v, buf_v, svec_v, sem0, sem1):
    wid = lax.axis_index("s") * 2 + lax.axis_index("c")
    base = wid * RPW
    tables = (a_hbm, bt_hbm, c_hbm)

    for d in range(NDIMS):
        pltpu.sync_copy(xT_hbm.at[d, pl.ds(base, RPW)],
                        xall_v.at[d, pl.ds(0, RPW)])
    pltpu.sync_copy(bias_hbm, b_v)

    iota = lax.iota(jnp.int32, L)

    for t in range(6):
        for kc in range(RPW // L):
            xc = xall_v[DMAP[t], pl.ds(kc * L, L)]
            idx_v[pl.ds(t * RPW + kc * L, L)] = xc * MUL[t] + ADD[t]

    tailmask = iota >= (L - (DIM - NFULL * L))
    zero16 = jnp.zeros((L,), jnp.float32)

    def issue(w, pb, sem):
        for t in range(6):
            pltpu.async_copy(
                tables[AMAP[t]].at[idx_v.at[pl.ds(t * RPW + w * RW, RW)]],
                buf_v.at[pb, t], sem)

    def drain(w, pb, sem):
        for t in range(6):
            pltpu.make_async_copy(
                tables[AMAP[t]].at[idx_v.at[pl.ds(t * RPW + w * RW, RW)]],
                buf_v.at[pb, t], sem).wait()

    def compute(w, pb):
        def chunk(off, accs, masked):
            b1 = b_v[pl.ds(DIM + off, L)]
            b2 = b_v[pl.ds(2 * DIM + off, L)]
            b3 = b_v[pl.ds(3 * DIM + off, L)]
            new = []
            for r in range(RW):
                a1 = buf_v[pb, 0, r, pl.ds(off, L)]
                a2 = buf_v[pb, 1, r, pl.ds(off, L)]
                a3 = buf_v[pb, 2, r, pl.ds(off, L)]
                f2 = buf_v[pb, 3, r, pl.ds(off, L)]
                f3 = buf_v[pb, 4, r, pl.ds(off, L)]
                g3 = buf_v[pb, 5, r, pl.ds(off, L)]
                e1 = jnp.exp(a1 + b1)
                e2 = jnp.exp(a2 + f2 + b2)
                e3 = jnp.exp(a3 + f3 + g3 + b3)
                if masked:
                    e1 = jnp.where(tailmask, e1, 0.0)
                    e2 = jnp.where(tailmask, e2, 0.0)
                    e3 = jnp.where(tailmask, e3, 0.0)
                s1, s2, s3 = accs[3 * r:3 * r + 3]
                new += [s1 + e1, s2 + e2, s3 + e3]
            return tuple(new)

        accs = lax.fori_loop(
            0, NFULL,
            lambda c, a: chunk(c * L, a, False),
            tuple([zero16] * (3 * RW)),
        )
        accs = chunk(TAIL_OFF, accs, True)

        for r in range(RW):
            row = w * RW + r
            xsc = [None] * NDIMS
            for j in range(NDIMS):
                xv = xall_v[j, pl.ds(row, L)]
                xsc[j] = xv[0]
            sel = zero16
            for t, j in PICKS:
                col = xsc[j]
                rem = lax.rem(col, L)
                v = buf_v[pb, t, r, pl.ds(col - rem, L)]
                sel = sel + jnp.where(iota == rem, v, 0.0)
            for j in range(NDIMS):
                off = j * DIM + xsc[j]
                rem = lax.rem(off, L)
                v = b_v[pl.ds(off - rem, L)]
                sel = sel + jnp.where(iota == rem, v, 0.0)
            svec_v[pl.ds(row * 64, L)] = sel
            for j in range(3):
                svec_v[pl.ds(row * 64 + (j + 1) * L, L)] = accs[3 * r + j]

    issue(0, 0, sem0)

    def body2(i, carry):
        w = 2 * i
        issue(w + 1, 1, sem1)
        drain(w, 0, sem0)
        compute(w, 0)
        issue(w + 2, 0, sem0)
        drain(w + 1, 1, sem1)
        compute(w + 1, 1)
        return carry

    lax.fori_loop(0, NWAVE // 2 - 1, body2, 0)
    issue(NWAVE - 1, 1, sem1)
    drain(NWAVE - 2, 0, sem0)
    compute(NWAVE - 2, 0)
    drain(NWAVE - 1, 1, sem1)
    compute(NWAVE - 1, 1)

    pltpu.sync_copy(svec_v, out_hbm.at[pl.ds(wid * (RPW * 64), RPW * 64)])


BT = 256


def _tc_kernel(svecs_ref, b_ref, out_ref):
    sv = svecs_ref[...]
    sel = jnp.sum(sv[:, 0:16], axis=1, keepdims=True)
    s1 = jnp.sum(sv[:, 16:32], axis=1, keepdims=True)
    s2 = jnp.sum(sv[:, 32:48], axis=1, keepdims=True)
    s3 = jnp.sum(sv[:, 48:64], axis=1, keepdims=True)
    b0 = b_ref[...][:, 0:DIM]
    s0 = jnp.sum(jnp.exp(b0))
    out_ref[...] = (sel - jnp.log(s0) - jnp.log(s1)
                    - jnp.log(s2) - jnp.log(s3))


@jax.jit
def kernel(x, W, b):
    x32 = x.astype(jnp.int32)
    xT = x32.T
    A = W[0:DIM, DIM:4 * DIM].reshape(3 * DIM, DIM)
    Bt = W[DIM:2 * DIM, 2 * DIM:4 * DIM].reshape(2 * DIM, DIM)
    C = W[2 * DIM:3 * DIM, 3 * DIM:4 * DIM]

    mesh = plsc.VectorSubcoreMesh(core_axis_name="c", subcore_axis_name="s")
    svecs = pl.kernel(
        _sc_kernel,
        mesh=mesh,
        out_type=[jax.ShapeDtypeStruct((BATCH * 64,), jnp.float32)],
        scratch_types=[
            pltpu.VMEM((NDIMS, RPW + L), jnp.int32),
            pltpu.VMEM((6 * RPW,), jnp.int32),
            pltpu.VMEM((NDIMS * DIM,), jnp.float32),
            pltpu.VMEM((2, 6, RW, DIM), jnp.float32),
            pltpu.VMEM((RPW * 64,), jnp.float32),
            pltpu.SemaphoreType.DMA,
            pltpu.SemaphoreType.DMA,
        ],
        compiler_params=pltpu.CompilerParams(use_tc_tiling_on_sc=False,
                                             needs_layout_passes=False),
    )(xT, A, Bt, C, b)[0]

    out = pl.pallas_call(
        _tc_kernel,
        grid=(BATCH // BT,),
        in_specs=[
            pl.BlockSpec((BT, 64), lambda i: (i, 0)),
            pl.BlockSpec((1, NDIMS * DIM), lambda i: (0, 0)),
        ],
        out_specs=pl.BlockSpec((BT, 1), lambda i: (i, 0)),
        out_shape=jax.ShapeDtypeStruct((BATCH, 1), jnp.float32),
    )(svecs.reshape(BATCH, 64), b.reshape(1, NDIMS * DIM))

    return out.reshape(BATCH)
